# Initial kernel scaffold; baseline (speedup 1.0000x reference)
import jax
import jax.numpy as jnp
from jax.experimental import pallas as pl


def kernel(x, w_packed, b_packed):
    raise NotImplementedError("write your pallas kernel here")


# speedup vs baseline: 1.3973x; 1.3973x over previous
import jax
import jax.numpy as jnp
from jax.experimental import pallas as pl
from jax.experimental.pallas import tpu as pltpu

IN_F = 10
OUT_F = 5
TB = 8192


def _round_up(n: int, m: int) -> int:
    return ((n + m - 1) // m) * m


def _linear_narrow_kernel(x_ref, w_ref, b_ref, o_ref):
    acc = jnp.dot(x_ref[...], w_ref[...], preferred_element_type=jnp.float32)
    o_ref[...] = (acc + b_ref[...]).astype(o_ref.dtype)


@jax.jit
def _forward(x, w_packed, b_packed):
    B, in_f = x.shape
    assert in_f == IN_F

    w = w_packed[:, :OUT_F]
    b = b_packed[:, :OUT_F]

    tb = min(TB, _round_up(B, 8))
    b_pad = _round_up(B, tb)
    if b_pad != B:
        x = jnp.pad(x, ((0, b_pad - B), (0, 0)))

    out = pl.pallas_call(
        _linear_narrow_kernel,
        out_shape=jax.ShapeDtypeStruct((b_pad, OUT_F), x.dtype),
        grid=(b_pad // tb,),
        in_specs=[
            pl.BlockSpec((tb, IN_F), lambda i: (i, 0)),
            pl.BlockSpec((IN_F, OUT_F), lambda i: (0, 0)),
            pl.BlockSpec((1, OUT_F), lambda i: (0, 0)),
        ],
        out_specs=pl.BlockSpec((tb, OUT_F), lambda i: (i, 0)),
        compiler_params=pltpu.CompilerParams(
            dimension_semantics=("parallel",),
        ),
    )(x, w, b)

    return out[:B] if b_pad != B else out


def kernel(x, w_packed, b_packed):
    return _forward(x, w_packed, b_packed)


# speedup vs baseline: 1.4032x; 1.0042x over previous
import jax
import jax.numpy as jnp
from jax.experimental import pallas as pl
from jax.experimental.pallas import tpu as pltpu

IN_F = 10
OUT_F = 5
TB = 16384


def _round_up(n: int, m: int) -> int:
    return ((n + m - 1) // m) * m


def _linear_narrow_kernel(x_ref, w_ref, b_ref, o_ref):
    acc = jnp.dot(x_ref[...], w_ref[...], preferred_element_type=jnp.float32)
    o_ref[...] = (acc + b_ref[...]).astype(o_ref.dtype)


@jax.jit
def _forward(x, w_packed, b_packed):
    B, in_f = x.shape
    assert in_f == IN_F

    w = w_packed[:, :OUT_F]
    b = b_packed[:, :OUT_F]

    tb = min(TB, _round_up(B, 8))
    b_pad = _round_up(B, tb)
    if b_pad != B:
        x = jnp.pad(x, ((0, b_pad - B), (0, 0)))

    out = pl.pallas_call(
        _linear_narrow_kernel,
        out_shape=jax.ShapeDtypeStruct((b_pad, OUT_F), x.dtype),
        grid=(b_pad // tb,),
        in_specs=[
            pl.BlockSpec((tb, IN_F), lambda i: (i, 0)),
            pl.BlockSpec((IN_F, OUT_F), lambda i: (0, 0)),
            pl.BlockSpec((1, OUT_F), lambda i: (0, 0)),
        ],
        out_specs=pl.BlockSpec((tb, OUT_F), lambda i: (i, 0)),
        compiler_params=pltpu.CompilerParams(
            dimension_semantics=("parallel",),
        ),
    )(x, w, b)

    return out[:B] if b_pad != B else out


def kernel(x, w_packed, b_packed):
    return _forward(x, w_packed, b_packed)
